# baseline (speedup 1.0000x reference)
import functools

import jax
import jax.numpy as jnp
from jax import lax
from jax.experimental import pallas as pl
from jax.experimental.pallas import tpu as pltpu
from jax.experimental.pallas import tpu_sc as plsc

N = 10000
E = 320000
H = 128

NUM_CORES = 2
NUM_SUBCORES = 16
NUM_WORKERS = NUM_CORES * NUM_SUBCORES
EDGES_PER_WORKER = E // NUM_WORKERS
CHUNK = 25
NUM_CHUNKS = EDGES_PER_WORKER // CHUNK
NBUF = 4
GROUPS = NUM_CHUNKS // NBUF
STRIPE = 624
LAST_STRIPE = N - STRIPE * (NUM_SUBCORES - 1)


def _uhg_weight(m):
  col = lax.broadcasted_iota(jnp.int32, m.shape, 1)
  m2 = jnp.where(col < 2, m, 0.0)
  sq = jnp.sum(m2 * m2, axis=1, keepdims=True)
  den = 1.0 - sq
  safe_den = jnp.maximum(jnp.abs(den), 1e-9)
  quad = sq / (safe_den * jnp.sign(den))
  return 1.0 / (quad + 1.0)


def _matmul_t(h, w_ref):
  return lax.dot_general(h, w_ref[...], (((1,), (1,)), ((), ())),
                         preferred_element_type=jnp.float32)


def _elu_bn(p_ref, scale_ref, shift_ref):
  h = p_ref[0] + p_ref[1]
  h = h * scale_ref[...] + shift_ref[...]
  return jnp.where(h > 0.0, h, jnp.exp(h) - 1.0)


def _pre_body(x_ref, w_ref, b_ref, mw_w_ref, mw_b_ref, t_ref, mw_ref):
  h = x_ref[...]
  t_ref[...] = _matmul_t(h, w_ref) + b_ref[...]
  m = _matmul_t(h, mw_w_ref) + mw_b_ref[...]
  mw_ref[...] = m * _uhg_weight(m)


def _mid_body(p_ref, scale_ref, shift_ref, w_ref, b_ref, mw_w_ref, mw_b_ref,
              t_ref, mw_ref):
  h = _elu_bn(p_ref, scale_ref, shift_ref)
  t_ref[...] = _matmul_t(h, w_ref) + b_ref[...]
  m = _matmul_t(h, mw_w_ref) + mw_b_ref[...]
  mw_ref[...] = m * _uhg_weight(m)


def _out_body(p_ref, scale_ref, shift_ref, wp_ref, bp_ref, out_ref):
  h = _elu_bn(p_ref, scale_ref, shift_ref)
  y = _matmul_t(h, wp_ref) + bp_ref[...]
  norm = jnp.maximum(jnp.sqrt(jnp.sum(y * y, axis=1, keepdims=True)), 1e-12)
  out_ref[...] = y / norm


_f32 = jnp.float32
_nh = jax.ShapeDtypeStruct((N, H), _f32)

_pre_call = pl.pallas_call(_pre_body, out_shape=(_nh, _nh))
_mid_call = pl.pallas_call(_mid_body, out_shape=(_nh, _nh))
_out_call = pl.pallas_call(_out_body, out_shape=_nh)


@functools.cache
def _make_sc_scatter():
  mesh = plsc.VectorSubcoreMesh(core_axis_name="c", subcore_axis_name="s",
                                num_cores=NUM_CORES,
                                num_subcores=NUM_SUBCORES)
  return functools.partial(
      pl.kernel,
      out_type=jax.ShapeDtypeStruct((NUM_CORES, N, H), _f32),
      mesh=mesh,
      scratch_types=[
          pltpu.VMEM((NBUF, 2, CHUNK), jnp.int32),
          pltpu.VMEM((NBUF, CHUNK, H), _f32),
          pltpu.VMEM_SHARED((N, H), _f32),
          pltpu.SemaphoreType.DMA((NBUF,)),
          pltpu.SemaphoreType.DMA((NBUF,)),
          pltpu.SemaphoreType.DMA((NBUF,)),
      ],
  )(_sc_scatter_body)


def _sc_scatter_body(mw_hbm, t_hbm, ei_hbm, out_hbm,
                     idx_v, rows_v, acc_sh, isem, gsem, ssem):
  c = lax.axis_index("c")
  s = lax.axis_index("s")
  w = s * NUM_CORES + c

  def _stripe(fn):
    @pl.when(s < NUM_SUBCORES - 1)
    def _():
      fn(pl.ds(s * STRIPE, STRIPE))

    @pl.when(s == NUM_SUBCORES - 1)
    def _():
      fn(pl.ds(STRIPE * (NUM_SUBCORES - 1), LAST_STRIPE))

  @pl.when(c == 0)
  def _():
    _stripe(lambda r: pltpu.sync_copy(mw_hbm.at[r], acc_sh.at[r]))

  @pl.when(c == 1)
  def _():
    _stripe(lambda r: pltpu.sync_copy(t_hbm.at[r], acc_sh.at[r]))

  def _idx_load(j, b):
    pltpu.async_copy(ei_hbm.at[w].at[j], idx_v.at[b], isem.at[b])

  def _idx_wait(j, b):
    pltpu.make_async_copy(ei_hbm.at[w].at[j], idx_v.at[b],
                          isem.at[b]).wait()

  def _gather(b):
    pltpu.async_copy(mw_hbm.at[idx_v.at[b].at[0]], rows_v.at[b], gsem.at[b])

  def _gather_wait(b):
    pltpu.make_async_copy(mw_hbm.at[idx_v.at[b].at[0]], rows_v.at[b],
                          gsem.at[b]).wait()

  def _scatter(b):
    pltpu.async_copy(rows_v.at[b], acc_sh.at[idx_v.at[b].at[1]], ssem.at[b],
                     add=True)

  def _scatter_wait(b):
    pltpu.make_async_copy(rows_v.at[b], acc_sh.at[idx_v.at[b].at[1]],
                          ssem.at[b]).wait()

  for b in range(NBUF):
    _idx_load(b, b)

  plsc.subcore_barrier()

  @pl.loop(0, GROUPS)
  def _edges(g):
    base = g * NBUF
    for b in range(NBUF):
      _idx_wait(base + b, b)
      _gather(b)
    for b in range(NBUF):
      _gather_wait(b)
      _scatter(b)
    for b in range(NBUF):
      _scatter_wait(b)

      @pl.when(g < GROUPS - 1)
      def _():
        _idx_load(base + NBUF + b, b)

  plsc.subcore_barrier()
  _stripe(lambda r: pltpu.sync_copy(acc_sh.at[r], out_hbm.at[c].at[r]))


def kernel(x, edge_index, W1, b1, M1w, M1b, g1, be1,
           W2, b2, M2w, M2b, g2, be2, Wp, bp):
  ei = edge_index.reshape(2, NUM_WORKERS, NUM_CHUNKS, CHUNK)
  ei = ei.transpose(1, 2, 0, 3)
  bn = 1.0 / jnp.sqrt(jnp.float32(1.0 + 1e-5))
  scale1 = (g1 * bn).reshape(1, H)
  shift1 = be1.reshape(1, H)
  scale2 = (g2 * bn).reshape(1, H)
  shift2 = be2.reshape(1, H)

  sc_scatter = _make_sc_scatter()
  t1, mw1 = _pre_call(x, W1, b1.reshape(1, H), M1w, M1b.reshape(1, H))
  parts1 = sc_scatter(mw1, t1, ei)
  t2, mw2 = _mid_call(parts1, scale1, shift1, W2, b2.reshape(1, H),
                      M2w, M2b.reshape(1, H))
  parts2 = sc_scatter(mw2, t2, ei)
  return _out_call(parts2, scale2, shift2, Wp, bp.reshape(1, H))


# speedup vs baseline: 1.3331x; 1.3331x over previous
import functools

import jax
import jax.numpy as jnp
from jax import lax
from jax.experimental import pallas as pl
from jax.experimental.pallas import tpu as pltpu
from jax.experimental.pallas import tpu_sc as plsc

N = 10000
E = 320000
H = 128

NUM_CORES = 2
NUM_SUBCORES = 16
NUM_WORKERS = NUM_CORES * NUM_SUBCORES
EDGES_PER_WORKER = E // NUM_WORKERS
CHUNK = 50
NUM_CHUNKS = EDGES_PER_WORKER // CHUNK
NBUF = 4
GROUPS = NUM_CHUNKS // NBUF
STRIPE = 624
LAST_STRIPE = N - STRIPE * (NUM_SUBCORES - 1)


def _uhg_weight(m):
  col = lax.broadcasted_iota(jnp.int32, m.shape, 1)
  m2 = jnp.where(col < 2, m, 0.0)
  sq = jnp.sum(m2 * m2, axis=1, keepdims=True)
  den = 1.0 - sq
  safe_den = jnp.maximum(jnp.abs(den), 1e-9)
  quad = sq / (safe_den * jnp.sign(den))
  return 1.0 / (quad + 1.0)


def _matmul_t(h, w_ref):
  return lax.dot_general(h, w_ref[...], (((1,), (1,)), ((), ())),
                         preferred_element_type=jnp.float32)


def _elu_bn(p_ref, scale_ref, shift_ref):
  h = p_ref[0] + p_ref[1]
  h = h * scale_ref[...] + shift_ref[...]
  return jnp.where(h > 0.0, h, jnp.exp(h) - 1.0)


def _pre_body(x_ref, w_ref, b_ref, mw_w_ref, mw_b_ref, t_ref, mw_ref):
  h = x_ref[...]
  t_ref[...] = _matmul_t(h, w_ref) + b_ref[...]
  m = _matmul_t(h, mw_w_ref) + mw_b_ref[...]
  mw_ref[...] = m * _uhg_weight(m)


def _mid_body(p_ref, scale_ref, shift_ref, w_ref, b_ref, mw_w_ref, mw_b_ref,
              t_ref, mw_ref):
  h = _elu_bn(p_ref, scale_ref, shift_ref)
  t_ref[...] = _matmul_t(h, w_ref) + b_ref[...]
  m = _matmul_t(h, mw_w_ref) + mw_b_ref[...]
  mw_ref[...] = m * _uhg_weight(m)


def _out_body(p_ref, scale_ref, shift_ref, wp_ref, bp_ref, out_ref):
  h = _elu_bn(p_ref, scale_ref, shift_ref)
  y = _matmul_t(h, wp_ref) + bp_ref[...]
  norm = jnp.maximum(jnp.sqrt(jnp.sum(y * y, axis=1, keepdims=True)), 1e-12)
  out_ref[...] = y / norm


_f32 = jnp.float32
_nh = jax.ShapeDtypeStruct((N, H), _f32)

_pre_call = pl.pallas_call(_pre_body, out_shape=(_nh, _nh))
_mid_call = pl.pallas_call(_mid_body, out_shape=(_nh, _nh))
_out_call = pl.pallas_call(_out_body, out_shape=_nh)


@functools.cache
def _make_sc_scatter():
  mesh = plsc.VectorSubcoreMesh(core_axis_name="c", subcore_axis_name="s",
                                num_cores=NUM_CORES,
                                num_subcores=NUM_SUBCORES)
  return functools.partial(
      pl.kernel,
      out_type=jax.ShapeDtypeStruct((NUM_CORES, N, H), _f32),
      mesh=mesh,
      scratch_types=[
          pltpu.VMEM((NBUF, 2, CHUNK), jnp.int32),
          pltpu.VMEM((NBUF, CHUNK, H), _f32),
          pltpu.VMEM_SHARED((N, H), _f32),
          pltpu.SemaphoreType.DMA((NBUF,)),
          pltpu.SemaphoreType.DMA((NBUF,)),
          pltpu.SemaphoreType.DMA((NBUF,)),
      ],
  )(_sc_scatter_body)


def _sc_scatter_body(mw_hbm, t_hbm, ei_hbm, out_hbm,
                     idx_v, rows_v, acc_sh, isem, gsem, ssem):
  c = lax.axis_index("c")
  s = lax.axis_index("s")
  w = s * NUM_CORES + c

  def _stripe(fn):
    @pl.when(s < NUM_SUBCORES - 1)
    def _():
      fn(pl.ds(s * STRIPE, STRIPE))

    @pl.when(s == NUM_SUBCORES - 1)
    def _():
      fn(pl.ds(STRIPE * (NUM_SUBCORES - 1), LAST_STRIPE))

  @pl.when(c == 0)
  def _():
    _stripe(lambda r: pltpu.sync_copy(mw_hbm.at[r], acc_sh.at[r]))

  @pl.when(c == 1)
  def _():
    _stripe(lambda r: pltpu.sync_copy(t_hbm.at[r], acc_sh.at[r]))

  def _idx_load(j, b):
    pltpu.async_copy(ei_hbm.at[w].at[j], idx_v.at[b], isem.at[b])

  def _idx_wait(j, b):
    pltpu.make_async_copy(ei_hbm.at[w].at[j], idx_v.at[b],
                          isem.at[b]).wait()

  def _gather(b):
    pltpu.async_copy(mw_hbm.at[idx_v.at[b].at[0]], rows_v.at[b], gsem.at[b])

  def _gather_wait(b):
    pltpu.make_async_copy(mw_hbm.at[idx_v.at[b].at[0]], rows_v.at[b],
                          gsem.at[b]).wait()

  def _scatter(b):
    pltpu.async_copy(rows_v.at[b], acc_sh.at[idx_v.at[b].at[1]], ssem.at[b],
                     add=True)

  def _scatter_wait(b):
    pltpu.make_async_copy(rows_v.at[b], acc_sh.at[idx_v.at[b].at[1]],
                          ssem.at[b]).wait()

  for b in range(NBUF):
    _idx_load(b, b)

  plsc.subcore_barrier()

  @pl.loop(0, GROUPS)
  def _edges(g):
    base = g * NBUF
    for b in range(NBUF):
      _idx_wait(base + b, b)
      _gather(b)
    for b in range(NBUF):
      _gather_wait(b)
      _scatter(b)
    for b in range(NBUF):
      _scatter_wait(b)

      @pl.when(g < GROUPS - 1)
      def _():
        _idx_load(base + NBUF + b, b)

  plsc.subcore_barrier()
  _stripe(lambda r: pltpu.sync_copy(acc_sh.at[r], out_hbm.at[c].at[r]))


def kernel(x, edge_index, W1, b1, M1w, M1b, g1, be1,
           W2, b2, M2w, M2b, g2, be2, Wp, bp):
  ei = edge_index.reshape(2, NUM_WORKERS, NUM_CHUNKS, CHUNK)
  ei = ei.transpose(1, 2, 0, 3)
  bn = 1.0 / jnp.sqrt(jnp.float32(1.0 + 1e-5))
  scale1 = (g1 * bn).reshape(1, H)
  shift1 = be1.reshape(1, H)
  scale2 = (g2 * bn).reshape(1, H)
  shift2 = be2.reshape(1, H)

  sc_scatter = _make_sc_scatter()
  t1, mw1 = _pre_call(x, W1, b1.reshape(1, H), M1w, M1b.reshape(1, H))
  parts1 = sc_scatter(mw1, t1, ei)
  t2, mw2 = _mid_call(parts1, scale1, shift1, W2, b2.reshape(1, H),
                      M2w, M2b.reshape(1, H))
  parts2 = sc_scatter(mw2, t2, ei)
  return _out_call(parts2, scale2, shift2, Wp, bp.reshape(1, H))
